# Initial kernel scaffold; baseline (speedup 1.0000x reference)
import jax
import jax.numpy as jnp
from jax.experimental import pallas as pl


def kernel(time, embeddings):
    raise NotImplementedError("write your pallas kernel here")


# speedup vs baseline: 1.4846x; 1.4846x over previous
import functools

import jax
import jax.numpy as jnp
from jax import lax
from jax.experimental import pallas as pl
from jax.experimental.pallas import tpu as pltpu
from jax.experimental.pallas import tpu_sc as plsc


@functools.lru_cache(maxsize=None)
def _make_gather(B: int, V: int, D: int, C: int):
    info = plsc.get_sparse_core_info()
    nc, ns = info.num_cores, info.num_subcores
    nw = nc * ns
    assert B % nw == 0
    b_per_w = B // nw
    assert b_per_w % C == 0
    n_chunks = b_per_w // C
    mesh = plsc.VectorSubcoreMesh(core_axis_name="c", subcore_axis_name="s")

    @functools.partial(
        pl.kernel,
        mesh=mesh,
        out_type=jax.ShapeDtypeStruct((B, D), jnp.float32),
        scratch_types=[
            pltpu.VMEM((b_per_w,), jnp.int32),
            pltpu.VMEM((C, D), jnp.float32),
            pltpu.SemaphoreType.DMA,
        ],
    )
    def k(time_hbm, table_hbm, out_hbm, idx_v, rows_v, sem):
        wid = lax.axis_index("s") * nc + lax.axis_index("c")
        base = wid * b_per_w
        pltpu.sync_copy(time_hbm.at[pl.ds(base, b_per_w)], idx_v)

        def body(c, carry):
            row0 = c * C
            pltpu.async_copy(
                table_hbm.at[idx_v.at[pl.ds(row0, C)]], rows_v, sem
            ).wait()
            pltpu.sync_copy(rows_v, out_hbm.at[pl.ds(base + row0, C)])
            return carry

        lax.fori_loop(0, n_chunks, body, 0)

    return k


def kernel(time, embeddings):
    (B,) = time.shape
    V, D = embeddings.shape
    return _make_gather(B, V, D, 16)(time.astype(jnp.int32), embeddings)


# speedup vs baseline: 1.6092x; 1.0839x over previous
import functools

import jax
import jax.numpy as jnp
from jax import lax
from jax.experimental import pallas as pl
from jax.experimental.pallas import tpu as pltpu
from jax.experimental.pallas import tpu_sc as plsc


@functools.lru_cache(maxsize=None)
def _make_gather(B: int, V: int, D: int, C: int):
    info = plsc.get_sparse_core_info()
    nc, ns = info.num_cores, info.num_subcores
    nw = nc * ns
    assert B % nw == 0
    b_per_w = B // nw
    assert b_per_w % (2 * C) == 0
    n_groups = b_per_w // (2 * C)
    mesh = plsc.VectorSubcoreMesh(core_axis_name="c", subcore_axis_name="s")

    @functools.partial(
        pl.kernel,
        mesh=mesh,
        out_type=jax.ShapeDtypeStruct((B, D), jnp.float32),
        scratch_types=[
            pltpu.VMEM((b_per_w,), jnp.int32),
            pltpu.VMEM((C, D), jnp.float32),
            pltpu.VMEM((C, D), jnp.float32),
            pltpu.SemaphoreType.DMA,
            pltpu.SemaphoreType.DMA,
            pltpu.SemaphoreType.DMA,
            pltpu.SemaphoreType.DMA,
        ],
    )
    def k(time_hbm, table_hbm, out_hbm, idx_v, buf0, buf1, gs0, gs1, ws0, ws1):
        wid = lax.axis_index("s") * nc + lax.axis_index("c")
        base = wid * b_per_w
        pltpu.sync_copy(time_hbm.at[pl.ds(base, b_per_w)], idx_v)
        bufs = (buf0, buf1)
        gsems = (gs0, gs1)
        wsems = (ws0, ws1)

        def gather(c, b):
            return pltpu.make_async_copy(
                table_hbm.at[idx_v.at[pl.ds(c * C, C)]], bufs[b], gsems[b]
            )

        def write(c, b):
            return pltpu.make_async_copy(
                bufs[b], out_hbm.at[pl.ds(base + c * C, C)], wsems[b]
            )

        gather(0, 0).start()

        def body(g, carry):
            c0 = 2 * g
            c1 = c0 + 1
            gather(c1, 1).start()
            gather(c0, 0).wait()
            write(c0, 0).start()
            gather(c1, 1).wait()
            write(c0, 0).wait()

            @pl.when(g + 1 < n_groups)
            def _():
                gather(c0 + 2, 0).start()

            write(c1, 1).start()
            write(c1, 1).wait()
            return carry

        lax.fori_loop(0, n_groups, body, 0)

    return k


def kernel(time, embeddings):
    (B,) = time.shape
    V, D = embeddings.shape
    return _make_gather(B, V, D, 8)(time.astype(jnp.int32), embeddings)
